# baseline (speedup 1.0000x reference)
import functools
import math

import jax
import jax.numpy as jnp
from jax import lax
from jax.experimental import pallas as pl
from jax.experimental.pallas import tpu as pltpu
from jax.experimental.pallas import tpu_sc as plsc

EMB_DIM = 64
D_MODEL = 768
SCALE = math.sqrt(EMB_DIM)

NC = 2
NS = 16
NW = NC * NS

CHUNK = 128


def _sc_gather(table, idx3d, n_chunks, b_per_w):
    mesh = plsc.VectorSubcoreMesh(core_axis_name="c", subcore_axis_name="s")

    @functools.partial(
        pl.kernel,
        mesh=mesh,
        out_type=jax.ShapeDtypeStruct((NW * b_per_w, EMB_DIM), jnp.float32),
        scratch_types=[
            pltpu.VMEM((n_chunks, CHUNK), jnp.int32),
            pltpu.VMEM((CHUNK, EMB_DIM), jnp.float32),
            pltpu.SemaphoreType.DMA,
        ],
        compiler_params=pltpu.CompilerParams(use_tc_tiling_on_sc=False),
    )
    def gather_kernel(idx_hbm, table_hbm, out_hbm, idx_v, rows_v, sem):
        wid = lax.axis_index("s") * NC + lax.axis_index("c")
        base = wid * b_per_w
        pltpu.sync_copy(idx_hbm.at[wid], idx_v)

        def body(c, carry):
            pltpu.async_copy(table_hbm.at[idx_v.at[c]], rows_v, sem).wait()
            pltpu.sync_copy(
                rows_v, out_hbm.at[pl.ds(base + c * CHUNK, CHUNK)]
            )
            return carry

        lax.fori_loop(0, n_chunks, body, 0)

    return gather_kernel(idx3d, table)


def _mm_body(x_ref, w_ref, b_ref, o_ref):
    x = x_ref[...] * SCALE
    acc = lax.dot_general(
        x, w_ref[...], (((1,), (1,)), ((), ())),
        preferred_element_type=jnp.float32,
    )
    o_ref[...] = acc + b_ref[...]


def _tc_project(gathered, proj_w, proj_b, block_m):
    n = gathered.shape[0]
    grid = (n // block_m,)
    return pl.pallas_call(
        _mm_body,
        grid=grid,
        in_specs=[
            pl.BlockSpec((block_m, EMB_DIM), lambda i: (i, 0)),
            pl.BlockSpec((D_MODEL, EMB_DIM), lambda i: (0, 0)),
            pl.BlockSpec((1, D_MODEL), lambda i: (0, 0)),
        ],
        out_specs=pl.BlockSpec((block_m, D_MODEL), lambda i: (i, 0)),
        out_shape=jax.ShapeDtypeStruct((n, D_MODEL), jnp.float32),
    )(gathered, proj_w, proj_b.reshape(1, D_MODEL))


def kernel(persona, emb_table, proj_w, proj_b):
    batch, hist = persona.shape
    n = batch * hist
    b_per_w = n // NW
    n_chunks = b_per_w // CHUNK
    idx3d = persona.reshape(NW, n_chunks, CHUNK).astype(jnp.int32)
    gathered = _sc_gather(emb_table, idx3d, n_chunks, b_per_w)
    out2d = _tc_project(gathered, proj_w, proj_b, block_m=1024)
    return out2d.reshape(batch, hist, D_MODEL)


# speedup vs baseline: 1.7300x; 1.7300x over previous
import functools
import math

import jax
import jax.numpy as jnp
from jax import lax
from jax.experimental import pallas as pl
from jax.experimental.pallas import tpu as pltpu
from jax.experimental.pallas import tpu_sc as plsc

EMB_DIM = 64
D_MODEL = 768
SCALE = math.sqrt(EMB_DIM)

NC = 2
NS = 16
NW = NC * NS

CHUNK = 128
L = 16


def _sc_gather(table2, idx2d, b_per_w):
    n_chunks = b_per_w // CHUNK
    mesh = plsc.VectorSubcoreMesh(core_axis_name="c", subcore_axis_name="s")

    @functools.partial(
        pl.kernel,
        mesh=mesh,
        out_type=jax.ShapeDtypeStruct((NW * b_per_w, 2 * EMB_DIM), jnp.float32),
        scratch_types=[
            pltpu.VMEM((b_per_w,), jnp.int32),
            pltpu.VMEM((b_per_w,), jnp.int32),
            pltpu.VMEM((CHUNK, 2 * EMB_DIM), jnp.float32),
            pltpu.SemaphoreType.DMA,
        ],
        compiler_params=pltpu.CompilerParams(needs_layout_passes=False),
    )
    def gather_kernel(idx_hbm, table_hbm, out_hbm, idx_v, q_v, rows_v, sem):
        wid = lax.axis_index("s") * NC + lax.axis_index("c")
        base = wid * b_per_w
        pltpu.sync_copy(idx_hbm.at[wid], idx_v)

        def qbody(i, carry):
            v = idx_v[pl.ds(i * L, L)]
            q_v[pl.ds(i * L, L)] = jax.lax.shift_right_logical(v, 1)
            return carry

        lax.fori_loop(0, b_per_w // L, qbody, 0)

        iota16 = lax.iota(jnp.int32, L)
        zero16 = jnp.zeros((L,), jnp.float32)

        def body(c, carry):
            pltpu.async_copy(
                table_hbm.at[q_v.at[pl.ds(c * CHUNK, CHUNK)]], rows_v, sem
            ).wait()

            def zbody(g, carry2):
                vi = idx_v[pl.ds(c * CHUNK + g * L, L)]
                colbase = EMB_DIM - (vi & 1) * EMB_DIM
                rowid = g * L + iota16
                for m in range(EMB_DIM):
                    plsc.store_scatter(rows_v, [rowid, colbase + m], zero16)
                return carry2

            lax.fori_loop(0, CHUNK // L, zbody, 0)
            pltpu.sync_copy(
                rows_v, out_hbm.at[pl.ds(base + c * CHUNK, CHUNK)]
            )
            return carry

        lax.fori_loop(0, n_chunks, body, 0)

    return gather_kernel(idx2d, table2)


def _mm_body(x_ref, w_ref, b_ref, o_ref):
    acc = jnp.dot(x_ref[...], w_ref[...], preferred_element_type=jnp.float32)
    o_ref[...] = acc + b_ref[...]


def _tc_project(gathered, w2, b2, block_m):
    n = gathered.shape[0]
    return pl.pallas_call(
        _mm_body,
        grid=(n // block_m,),
        in_specs=[
            pl.BlockSpec((block_m, 2 * EMB_DIM), lambda i: (i, 0)),
            pl.BlockSpec((2 * EMB_DIM, D_MODEL), lambda i: (0, 0)),
            pl.BlockSpec((1, D_MODEL), lambda i: (0, 0)),
        ],
        out_specs=pl.BlockSpec((block_m, D_MODEL), lambda i: (i, 0)),
        out_shape=jax.ShapeDtypeStruct((n, D_MODEL), jnp.float32),
    )(gathered, w2, b2)


def kernel(persona, emb_table, proj_w, proj_b):
    batch, hist = persona.shape
    n = batch * hist
    b_per_w = n // NW
    table2 = emb_table.reshape(emb_table.shape[0] // 2, 2 * EMB_DIM)
    idx2d = persona.astype(jnp.int32).T.reshape(NW, b_per_w)
    gathered = _sc_gather(table2, idx2d, b_per_w)
    wt8 = jnp.transpose(proj_w) * SCALE
    w2 = jnp.concatenate([wt8, wt8], axis=0)
    out2d = _tc_project(gathered, w2, proj_b.reshape(1, D_MODEL), 1024)
    return out2d.reshape(hist, batch, D_MODEL).transpose(1, 0, 2)
